# baseline (speedup 1.0000x reference)
import functools

import jax
import jax.numpy as jnp
from jax import lax
from jax.experimental import pallas as pl
from jax.experimental.pallas import tpu as pltpu
from jax.experimental.pallas import tpu_sc as plsc

NUM_EMB = 1_000_000
DIM = 64
B_TOK = 16384
SEQ = 50
NC, NS = 2, 16
NW = NC * NS
BW = B_TOK // NW
CHUNK = 128
BB = 256
NB_HI = B_TOK // CHUNK


def _sc_gather_t(idx3, table2):
    mesh = plsc.VectorSubcoreMesh(core_axis_name="c", subcore_axis_name="s")

    @functools.partial(
        pl.kernel,
        mesh=mesh,
        out_type=jax.ShapeDtypeStruct((SEQ, DIM, B_TOK), jnp.float32),
        scratch_types=[
            pltpu.VMEM((SEQ, 4, CHUNK), jnp.int32),
            pltpu.VMEM((2, BB, DIM), jnp.float32),
            pltpu.VMEM((2, DIM, BB + 1), jnp.float32),
            pltpu.SemaphoreType.DMA,
            pltpu.SemaphoreType.DMA,
            pltpu.SemaphoreType.DMA,
            pltpu.SemaphoreType.DMA,
        ],
        compiler_params=pltpu.CompilerParams(
            use_tc_tiling_on_sc=False, needs_layout_passes=False),
    )
    def k(idx_hbm, table_hbm, out_hbm, idx_v, rows_v, tbuf,
          gsem0, gsem1, wsem0, wsem1):
        wid = lax.axis_index("s") * NC + lax.axis_index("c")
        gsems = (gsem0, gsem1)
        wsems = (wsem0, wsem1)

        pltpu.sync_copy(idx_hbm.at[:, pl.ds(4 * wid, 4), :], idx_v)

        def fire_gathers(s, h):
            for j in range(2):
                pltpu.async_copy(
                    table_hbm.at[idx_v.at[s, 2 * h + j]],
                    rows_v.at[h, pl.ds(j * CHUNK, CHUNK)], gsems[h])

        def drain_gathers(h):
            for j in range(2):
                pltpu.make_async_copy(
                    table_hbm.at[pl.ds(0, CHUNK)],
                    rows_v.at[h, pl.ds(j * CHUNK, CHUNK)], gsems[h]).wait()

        def transpose(h):
            rows = rows_v.at[h]
            tb = tbuf.at[h]
            lane = lax.iota(jnp.int32, 16)

            def rbody(rg, _):
                for i in range(16):
                    r = rg * 16 + i
                    rvec = jnp.zeros((16,), jnp.int32) + r
                    for kk in range(DIM // 16):
                        v = rows[r, pl.ds(kk * 16, 16)]
                        plsc.store_scatter(tb, [kk * 16 + lane, rvec], v)
                return 0

            lax.fori_loop(0, BB // 16, rbody, 0)

        def fire_write(s, h):
            boff = wid * BW + h * BB
            pltpu.async_copy(tbuf.at[h, :, pl.ds(0, BB)],
                             out_hbm.at[s, :, pl.ds(boff, BB)], wsems[h])

        def wait_write(h):
            pltpu.make_async_copy(
                tbuf.at[h, :, pl.ds(0, BB)],
                out_hbm.at[0, :, pl.ds(0, BB)], wsems[h]).wait()

        fire_gathers(0, 0)
        fire_gathers(0, 1)
        drain_gathers(0)
        transpose(0)
        fire_write(0, 0)
        fire_gathers(1, 0)
        drain_gathers(1)
        transpose(1)
        fire_write(0, 1)
        fire_gathers(1, 1)

        def body(s, _):
            for h in range(2):
                drain_gathers(h)
                transpose(h)
                wait_write(h)
                fire_write(s, h)
                fire_gathers(s + 1, h)
            return 0

        lax.fori_loop(1, SEQ - 1, body, 0)

        for h in range(2):
            drain_gathers(h)
            transpose(h)
            wait_write(h)
            fire_write(SEQ - 1, h)
        wait_write(0)
        wait_write(1)

    return k(idx3, table2)


def _tc_detranspose(emb_t):
    RB = 512

    def body(tin, tout):
        tout[...] = tin[...].T

    return pl.pallas_call(
        body,
        grid=(pl.cdiv(NUM_EMB, RB),),
        in_specs=[pl.BlockSpec((DIM, RB), lambda i: (0, i))],
        out_specs=pl.BlockSpec((RB, DIM), lambda i: (i, 0)),
        out_shape=jax.ShapeDtypeStruct((NUM_EMB, DIM), jnp.float32),
    )(emb_t)


def kernel(token_ids, emb):
    idx3 = token_ids.T.astype(jnp.int32).reshape(SEQ, NB_HI, CHUNK)
    table2 = _tc_detranspose(emb.T)
    out_t = _sc_gather_t(idx3, table2)
    return jnp.transpose(out_t, (2, 0, 1))


# speedup vs baseline: 2.2829x; 2.2829x over previous
import functools

import jax
import jax.numpy as jnp
from jax import lax
from jax.experimental import pallas as pl
from jax.experimental.pallas import tpu as pltpu
from jax.experimental.pallas import tpu_sc as plsc

NUM_EMB = 1_000_000
DIM = 64
B_TOK = 16384
SEQ = 50
NC, NS = 2, 16
NW = NC * NS
BW = B_TOK // NW
CHUNK = 128
BB = 256
NB_HI = B_TOK // CHUNK


def _sc_gather_t(idx3, table2):
    mesh = plsc.VectorSubcoreMesh(core_axis_name="c", subcore_axis_name="s")

    @functools.partial(
        pl.kernel,
        mesh=mesh,
        out_type=jax.ShapeDtypeStruct((SEQ, 8, NB_HI, 8, CHUNK), jnp.float32),
        scratch_types=[
            pltpu.VMEM((SEQ, 4, CHUNK), jnp.int32),
            pltpu.VMEM((2, BB, DIM), jnp.float32),
            pltpu.VMEM((2, DIM, BB + 1), jnp.float32),
            pltpu.SemaphoreType.DMA,
            pltpu.SemaphoreType.DMA,
            pltpu.SemaphoreType.DMA,
            pltpu.SemaphoreType.DMA,
        ],
        compiler_params=pltpu.CompilerParams(
            use_tc_tiling_on_sc=False, needs_layout_passes=False),
    )
    def k(idx_hbm, table_hbm, out_hbm, idx_v, rows_v, tbuf,
          gsem0, gsem1, wsem0, wsem1):
        wid = lax.axis_index("s") * NC + lax.axis_index("c")
        gsems = (gsem0, gsem1)
        wsems = (wsem0, wsem1)

        pltpu.sync_copy(idx_hbm.at[:, pl.ds(4 * wid, 4), :], idx_v)

        def fire_gathers(s, h):
            for j in range(2):
                pltpu.async_copy(
                    table_hbm.at[idx_v.at[s, 2 * h + j]],
                    rows_v.at[h, pl.ds(j * CHUNK, CHUNK)], gsems[h])

        def drain_gathers(h):
            for j in range(2):
                pltpu.make_async_copy(
                    table_hbm.at[pl.ds(0, CHUNK)],
                    rows_v.at[h, pl.ds(j * CHUNK, CHUNK)], gsems[h]).wait()

        def transpose(h):
            rows = rows_v.at[h]
            tb = tbuf.at[h]
            lane = lax.iota(jnp.int32, 16)

            def rbody(rg, _):
                for i in range(16):
                    r = rg * 16 + i
                    rvec = jnp.zeros((16,), jnp.int32) + r
                    for kk in range(DIM // 16):
                        v = rows[r, pl.ds(kk * 16, 16)]
                        plsc.store_scatter(tb, [kk * 16 + lane, rvec], v)
                return 0

            lax.fori_loop(0, BB // 16, rbody, 0)

        def fire_write(s, h):
            tj0 = wid * 4 + h * 2
            for ti in range(8):
                for tjl in range(2):
                    pltpu.async_copy(
                        tbuf.at[h, pl.ds(8 * ti, 8), pl.ds(tjl * CHUNK, CHUNK)],
                        out_hbm.at[s, ti, tj0 + tjl], wsems[h])

        def wait_write(h):
            for _ in range(16):
                pltpu.make_async_copy(
                    tbuf.at[h, pl.ds(0, 8), pl.ds(0, CHUNK)],
                    out_hbm.at[0, 0, 0], wsems[h]).wait()

        fire_gathers(0, 0)
        fire_gathers(0, 1)
        drain_gathers(0)
        transpose(0)
        fire_write(0, 0)
        fire_gathers(1, 0)
        drain_gathers(1)
        transpose(1)
        fire_write(0, 1)
        fire_gathers(1, 1)

        def body(s, _):
            for h in range(2):
                drain_gathers(h)
                transpose(h)
                wait_write(h)
                fire_write(s, h)
                fire_gathers(s + 1, h)
            return 0

        lax.fori_loop(1, SEQ - 1, body, 0)

        for h in range(2):
            drain_gathers(h)
            transpose(h)
            wait_write(h)
            fire_write(SEQ - 1, h)
        wait_write(0)
        wait_write(1)

    return k(idx3, table2)


def kernel(token_ids, emb):
    idx3 = (token_ids.T.astype(jnp.int32) * 2).reshape(SEQ, NB_HI, CHUNK)
    table2 = jnp.pad(emb, ((0, 0), (0, DIM))).reshape(2 * NUM_EMB, DIM)
    out_t = _sc_gather_t(idx3, table2)
    return jnp.transpose(out_t, (2, 4, 0, 1, 3)).reshape(B_TOK, SEQ, DIM)
